# Initial kernel scaffold; baseline (speedup 1.0000x reference)
import jax
import jax.numpy as jnp
from jax.experimental import pallas as pl


def kernel(x, backbone_params, rpn_w, rpn_b, cls_w, cls_b, reg_w, reg_b, mconv1_w, mdeconv_w, mfinal_w):
    raise NotImplementedError("write your pallas kernel here")


# speedup vs baseline: 3.0721x; 3.0721x over previous
import numpy as np
import jax
import jax.numpy as jnp
from jax import lax
from jax.experimental import pallas as pl
from jax.experimental.pallas import tpu as pltpu

_BF = jnp.bfloat16
_F32 = jnp.float32


def _conv(x, w, stride, pad):
    return lax.conv_general_dilated(x, w, (stride, stride), [(pad, pad), (pad, pad)],
                                    dimension_numbers=('NCHW', 'OIHW', 'NCHW'))


def _bn(x, p):
    inv = p['g'] / jnp.sqrt(p['v'] + 1e-5)
    return x * inv[None, :, None, None] + (p['b'] - p['m'] * inv)[None, :, None, None]


def _resblock(x, p, stride):
    out = jax.nn.relu(_bn(_conv(x, p['c1'], stride, 1), p['b1']))
    out = _bn(_conv(out, p['c2'], 1, 1), p['b2'])
    idn = _bn(_conv(x, p['dc'], stride, 0), p['db']) if 'dc' in p else x
    return jax.nn.relu(out + idn)


def _backbone(x, P):
    x = jax.nn.relu(_bn(_conv(x, P['conv1'], 2, 3), P['bn1']))
    x = lax.reduce_window(x, -jnp.inf, lax.max, (1, 1, 3, 3), (1, 1, 2, 2),
                          [(0, 0), (0, 0), (1, 1), (1, 1)])
    for name, s in (('l1b1', 1), ('l1b2', 1), ('l2b1', 2), ('l2b2', 1), ('l3b1', 2), ('l3b2', 1)):
        x = _resblock(x, P[name], s)
    return x


def _anchor_grid_np():
    base, scale = 16, 2
    g = 224 // base
    ys, xs = np.meshgrid(np.arange(g), np.arange(g), indexing='ij')
    xc = base // 2 + xs * base
    yc = base // 2 + ys * base
    w = h = int(base * scale)
    a = np.stack([xc - w / 2, yc - h / 2, xc + w / 2, yc + h / 2], -1).astype(np.float32)
    return np.clip(a - 1, 0, None).reshape(-1, 4)


def _rpn_body(fm_ref, anc_ref, w1_ref, b1_ref, cw_ref, cb_ref, rw_ref, rb_ref,
              sc_ref, dl_ref, bx_ref, pad_ref):
    R = 1568
    pad_ref[0:16, :] = jnp.zeros((16, 256), _BF)
    pad_ref[R + 16:R + 32, :] = jnp.zeros((16, 256), _BF)
    pad_ref[16:R + 16, :] = fm_ref[...]
    r = lax.broadcasted_iota(jnp.int32, (R, 1), 0)
    loc = r % 196
    w_i = loc // 14
    h_i = loc % 14
    acc = jnp.zeros((R, 32), _F32)
    for k in range(9):
        kh, kw = k // 3, k % 3
        dh, dw = kh - 1, kw - 1
        s = dw * 14 + dh
        ok = (h_i + dh >= 0) & (h_i + dh <= 13) & (w_i + dw >= 0) & (w_i + dw <= 13)
        x = jnp.where(ok, pad_ref[16 + s:16 + s + R, :], jnp.zeros((), _BF))
        acc = acc + jnp.dot(x, w1_ref[k], preferred_element_type=_F32)
    rpn = jax.nn.relu(acc + b1_ref[...]).astype(_BF)
    sc = jnp.dot(rpn, cw_ref[...], preferred_element_type=_F32) + cb_ref[...]
    dl = jnp.dot(rpn, rw_ref[...], preferred_element_type=_F32) + rb_ref[...]
    sc_ref[0] = sc
    dl_ref[0] = dl
    anc = anc_ref[...]
    x_a = anc[:, 0:1]
    y_a = anc[:, 1:2]
    w_a = anc[:, 2:3] - x_a
    h_a = anc[:, 3:4] - y_a
    bx1 = dl[:, 0:1] * w_a + x_a
    by1 = dl[:, 1:2] * h_a + y_a
    bx2 = bx1 + jnp.exp(dl[:, 2:3]) * w_a
    by2 = by1 + jnp.exp(dl[:, 3:4]) * h_a
    b = jnp.concatenate([by1, bx1, by2, bx2], axis=1)
    bx_ref[0] = jnp.floor(jnp.clip(b, 0.0, 223.0)) / 223.0


def _mask_body(fm_ref, box_ref, w1_ref, wd_ref, wf_ref, out_ref, pad_ref):
    N = 98
    M = N * 64
    bx = box_ref[0].reshape(N, 1, 4)
    q = lax.broadcasted_iota(jnp.int32, (N, 8, 196), 2)
    hq = (q // 14).astype(_F32)
    wq = (q % 14).astype(_F32)
    ii = lax.broadcasted_iota(jnp.int32, (N, 8, 196), 1).astype(_F32)

    def taps(c1, c2, grid_f):
        t = c1 * 13.0 + ii * (c2 - c1) * 13.0 / 6.0
        t0 = jnp.floor(t)
        lt = t - t0
        wt = (jnp.where(grid_f == t0, 1.0 - lt, 0.0)
              + jnp.where(grid_f == jnp.ceil(t), lt, 0.0))
        return jnp.where(ii < 7.0, wt, 0.0)

    wy = taps(bx[:, :, 0:1], bx[:, :, 2:3], hq)
    wx = taps(bx[:, :, 1:2], bx[:, :, 3:4], wq)
    bigk = (wy[:, :, None, :] * wx[:, None, :, :]).astype(_BF).reshape(M, 196)
    crop = jnp.dot(bigk, fm_ref[0], preferred_element_type=_F32)
    pad_ref[0:16, :] = jnp.zeros((16, 256), _BF)
    pad_ref[M + 16:M + 32, :] = jnp.zeros((16, 256), _BF)
    pad_ref[16:M + 16, :] = crop.astype(_BF)
    acc = jnp.zeros((M, 256), _F32)
    for k in range(9):
        s = (k // 3 - 1) * 8 + (k % 3 - 1)
        acc = acc + jnp.dot(pad_ref[16 + s:16 + s + M, :], w1_ref[k],
                            preferred_element_type=_F32)
    xb = jax.nn.relu(acc).astype(_BF)
    zs = []
    for dij in range(4):
        h = jnp.dot(xb, wd_ref[dij], preferred_element_type=_F32)
        hb = jax.nn.relu(h).astype(_BF)
        z = jnp.dot(hb, wf_ref[...], preferred_element_type=_F32)
        zs.append(1.0 / (1.0 + jnp.exp(-z)))
    out_ref[0] = jnp.concatenate(zs, axis=1)


def kernel(x, backbone_params, rpn_w, rpn_b, cls_w, cls_b, reg_w, reg_b,
           mconv1_w, mdeconv_w, mfinal_w):
    B = x.shape[0]
    fmap = _backbone(x, backbone_params)

    f_wh = fmap.transpose(0, 3, 2, 1).reshape(B * 196, 256).astype(_BF)
    f_hw = fmap.transpose(0, 2, 3, 1).reshape(B, 196, 256).astype(_BF)

    anchors = jnp.asarray(_anchor_grid_np())
    anc_big = jnp.tile(anchors, (8, 1))

    rpn_wt = rpn_w.transpose(2, 3, 1, 0).reshape(9, 256, 32).astype(_BF)
    cls_wt = cls_w[:, :, 0, 0].T.astype(_BF)
    reg_wt = reg_w[:, :, 0, 0].T.astype(_BF)

    sc, dl, bxo = pl.pallas_call(
        _rpn_body,
        grid=(2,),
        in_specs=[
            pl.BlockSpec((1568, 256), lambda g: (g, 0)),
            pl.BlockSpec((1568, 4), lambda g: (0, 0)),
            pl.BlockSpec((9, 256, 32), lambda g: (0, 0, 0)),
            pl.BlockSpec((1, 32), lambda g: (0, 0)),
            pl.BlockSpec((32, 2), lambda g: (0, 0)),
            pl.BlockSpec((1, 2), lambda g: (0, 0)),
            pl.BlockSpec((32, 4), lambda g: (0, 0)),
            pl.BlockSpec((1, 4), lambda g: (0, 0)),
        ],
        out_specs=[
            pl.BlockSpec((1, 1568, 2), lambda g: (g, 0, 0)),
            pl.BlockSpec((1, 1568, 4), lambda g: (g, 0, 0)),
            pl.BlockSpec((1, 1568, 4), lambda g: (g, 0, 0)),
        ],
        out_shape=[
            jax.ShapeDtypeStruct((2, 1568, 2), _F32),
            jax.ShapeDtypeStruct((2, 1568, 4), _F32),
            jax.ShapeDtypeStruct((2, 1568, 4), _F32),
        ],
        scratch_shapes=[pltpu.VMEM((1600, 256), _BF)],
        compiler_params=pltpu.CompilerParams(
            dimension_semantics=("parallel",),
        ),
        name="rpn_decode",
    )(f_wh, anc_big, rpn_wt, rpn_b.reshape(1, 32),
      cls_wt, cls_b.reshape(1, 2), reg_wt, reg_b.reshape(1, 4))

    scores = sc.reshape(B, 196, 2)
    deltas = dl.reshape(B, 196, 4)
    boxes32 = bxo.reshape(2 * B, 98, 4)

    w1t = mconv1_w.transpose(2, 3, 1, 0).reshape(9, 256, 256).astype(_BF)
    wdt = mdeconv_w.transpose(2, 3, 0, 1).reshape(4, 256, 256).astype(_BF)
    wft = mfinal_w[:, :, 0, 0].T.astype(_BF)

    masks8 = pl.pallas_call(
        _mask_body,
        grid=(2 * B,),
        in_specs=[
            pl.BlockSpec((1, 196, 256), lambda t: (t // 2, 0, 0)),
            pl.BlockSpec((1, 98, 4), lambda t: (t, 0, 0)),
            pl.BlockSpec((9, 256, 256), lambda t: (0, 0, 0)),
            pl.BlockSpec((4, 256, 256), lambda t: (0, 0, 0)),
            pl.BlockSpec((256, 2), lambda t: (0, 0)),
        ],
        out_specs=pl.BlockSpec((1, 6272, 8), lambda t: (t, 0, 0)),
        out_shape=jax.ShapeDtypeStruct((2 * B, 6272, 8), _F32),
        scratch_shapes=[pltpu.VMEM((6304, 256), _BF)],
        compiler_params=pltpu.CompilerParams(
            dimension_semantics=("parallel",),
            vmem_limit_bytes=56 * 1024 * 1024,
        ),
        name="roialign_mask_head",
    )(f_hw, boxes32, w1t, wdt, wft)

    m = masks8.reshape(B, 2, 98, 8, 8, 2, 2, 2)[:, :, :, :7, :7]
    masks = m.transpose(0, 1, 2, 7, 3, 5, 4, 6).reshape(B, 196, 2, 14, 14)
    return (scores, deltas, anchors, masks)


# speedup vs baseline: 3.0881x; 1.0052x over previous
import numpy as np
import jax
import jax.numpy as jnp
from jax import lax
from jax.experimental import pallas as pl
from jax.experimental.pallas import tpu as pltpu

_BF = jnp.bfloat16
_F32 = jnp.float32


def _conv(x, w, stride, pad):
    return lax.conv_general_dilated(x.astype(_BF), w.astype(_BF), (stride, stride),
                                    [(pad, pad), (pad, pad)],
                                    dimension_numbers=('NCHW', 'OIHW', 'NCHW'),
                                    preferred_element_type=_F32)


def _bn(x, p):
    inv = p['g'] / jnp.sqrt(p['v'] + 1e-5)
    return x * inv[None, :, None, None] + (p['b'] - p['m'] * inv)[None, :, None, None]


def _resblock(x, p, stride):
    out = jax.nn.relu(_bn(_conv(x, p['c1'], stride, 1), p['b1']))
    out = _bn(_conv(out, p['c2'], 1, 1), p['b2'])
    idn = _bn(_conv(x, p['dc'], stride, 0), p['db']) if 'dc' in p else x
    return jax.nn.relu(out + idn)


def _backbone(x, P):
    x = jax.nn.relu(_bn(_conv(x, P['conv1'], 2, 3), P['bn1']))
    x = lax.reduce_window(x, -jnp.inf, lax.max, (1, 1, 3, 3), (1, 1, 2, 2),
                          [(0, 0), (0, 0), (1, 1), (1, 1)])
    for name, s in (('l1b1', 1), ('l1b2', 1), ('l2b1', 2), ('l2b2', 1), ('l3b1', 2), ('l3b2', 1)):
        x = _resblock(x, P[name], s)
    return x


def _anchor_grid_np():
    base, scale = 16, 2
    g = 224 // base
    ys, xs = np.meshgrid(np.arange(g), np.arange(g), indexing='ij')
    xc = base // 2 + xs * base
    yc = base // 2 + ys * base
    w = h = int(base * scale)
    a = np.stack([xc - w / 2, yc - h / 2, xc + w / 2, yc + h / 2], -1).astype(np.float32)
    return np.clip(a - 1, 0, None).reshape(-1, 4)


def _rpn_body(fm_ref, anc_ref, w1_ref, b1_ref, cw_ref, cb_ref, rw_ref, rb_ref,
              sc_ref, dl_ref, bx_ref, pad_ref):
    R = 1568
    pad_ref[0:16, :] = jnp.zeros((16, 256), _BF)
    pad_ref[R + 16:R + 32, :] = jnp.zeros((16, 256), _BF)
    pad_ref[16:R + 16, :] = fm_ref[...]
    r = lax.broadcasted_iota(jnp.int32, (R, 1), 0)
    loc = r % 196
    w_i = loc // 14
    h_i = loc % 14
    acc = jnp.zeros((R, 32), _F32)
    for k in range(9):
        kh, kw = k // 3, k % 3
        dh, dw = kh - 1, kw - 1
        s = dw * 14 + dh
        ok = (h_i + dh >= 0) & (h_i + dh <= 13) & (w_i + dw >= 0) & (w_i + dw <= 13)
        x = jnp.where(ok, pad_ref[16 + s:16 + s + R, :], jnp.zeros((), _BF))
        acc = acc + jnp.dot(x, w1_ref[k], preferred_element_type=_F32)
    rpn = jax.nn.relu(acc + b1_ref[...]).astype(_BF)
    sc = jnp.dot(rpn, cw_ref[...], preferred_element_type=_F32) + cb_ref[...]
    dl = jnp.dot(rpn, rw_ref[...], preferred_element_type=_F32) + rb_ref[...]
    sc_ref[0] = sc
    dl_ref[0] = dl
    anc = anc_ref[...]
    x_a = anc[:, 0:1]
    y_a = anc[:, 1:2]
    w_a = anc[:, 2:3] - x_a
    h_a = anc[:, 3:4] - y_a
    bx1 = dl[:, 0:1] * w_a + x_a
    by1 = dl[:, 1:2] * h_a + y_a
    bx2 = bx1 + jnp.exp(dl[:, 2:3]) * w_a
    by2 = by1 + jnp.exp(dl[:, 3:4]) * h_a
    b = jnp.concatenate([by1, bx1, by2, bx2], axis=1)
    bx_ref[0] = jnp.floor(jnp.clip(b, 0.0, 223.0)) / 223.0


def _mask_body(fm_ref, box_ref, w1_ref, wd_ref, wf_ref, out_ref, pad_ref):
    N = 98
    M = N * 64
    bx = box_ref[0].reshape(N, 1, 4)
    q = lax.broadcasted_iota(jnp.int32, (N, 8, 196), 2)
    hq = (q // 14).astype(_F32)
    wq = (q % 14).astype(_F32)
    ii = lax.broadcasted_iota(jnp.int32, (N, 8, 196), 1).astype(_F32)

    def taps(c1, c2, grid_f):
        t = c1 * 13.0 + ii * (c2 - c1) * 13.0 / 6.0
        t0 = jnp.floor(t)
        lt = t - t0
        wt = (jnp.where(grid_f == t0, 1.0 - lt, 0.0)
              + jnp.where(grid_f == jnp.ceil(t), lt, 0.0))
        return jnp.where(ii < 7.0, wt, 0.0)

    wy = taps(bx[:, :, 0:1], bx[:, :, 2:3], hq)
    wx = taps(bx[:, :, 1:2], bx[:, :, 3:4], wq)
    bigk = (wy[:, :, None, :] * wx[:, None, :, :]).astype(_BF).reshape(M, 196)
    crop = jnp.dot(bigk, fm_ref[0], preferred_element_type=_F32)
    pad_ref[0:16, :] = jnp.zeros((16, 256), _BF)
    pad_ref[M + 16:M + 32, :] = jnp.zeros((16, 256), _BF)
    pad_ref[16:M + 16, :] = crop.astype(_BF)
    acc = jnp.zeros((M, 256), _F32)
    for k in range(9):
        s = (k // 3 - 1) * 8 + (k % 3 - 1)
        acc = acc + jnp.dot(pad_ref[16 + s:16 + s + M, :], w1_ref[k],
                            preferred_element_type=_F32)
    xb = jax.nn.relu(acc).astype(_BF)
    zs = []
    for dij in range(4):
        h = jnp.dot(xb, wd_ref[dij], preferred_element_type=_F32)
        hb = jax.nn.relu(h).astype(_BF)
        z = jnp.dot(hb, wf_ref[...], preferred_element_type=_F32)
        zs.append(1.0 / (1.0 + jnp.exp(-z)))
    out_ref[0] = jnp.concatenate(zs, axis=1)


def kernel(x, backbone_params, rpn_w, rpn_b, cls_w, cls_b, reg_w, reg_b,
           mconv1_w, mdeconv_w, mfinal_w):
    B = x.shape[0]
    fmap = _backbone(x, backbone_params)

    f_wh = fmap.transpose(0, 3, 2, 1).reshape(B * 196, 256).astype(_BF)
    f_hw = fmap.transpose(0, 2, 3, 1).reshape(B, 196, 256).astype(_BF)

    anchors = jnp.asarray(_anchor_grid_np())
    anc_big = jnp.tile(anchors, (8, 1))

    rpn_wt = rpn_w.transpose(2, 3, 1, 0).reshape(9, 256, 32).astype(_BF)
    cls_wt = cls_w[:, :, 0, 0].T.astype(_BF)
    reg_wt = reg_w[:, :, 0, 0].T.astype(_BF)

    sc, dl, bxo = pl.pallas_call(
        _rpn_body,
        grid=(2,),
        in_specs=[
            pl.BlockSpec((1568, 256), lambda g: (g, 0)),
            pl.BlockSpec((1568, 4), lambda g: (0, 0)),
            pl.BlockSpec((9, 256, 32), lambda g: (0, 0, 0)),
            pl.BlockSpec((1, 32), lambda g: (0, 0)),
            pl.BlockSpec((32, 2), lambda g: (0, 0)),
            pl.BlockSpec((1, 2), lambda g: (0, 0)),
            pl.BlockSpec((32, 4), lambda g: (0, 0)),
            pl.BlockSpec((1, 4), lambda g: (0, 0)),
        ],
        out_specs=[
            pl.BlockSpec((1, 1568, 2), lambda g: (g, 0, 0)),
            pl.BlockSpec((1, 1568, 4), lambda g: (g, 0, 0)),
            pl.BlockSpec((1, 1568, 4), lambda g: (g, 0, 0)),
        ],
        out_shape=[
            jax.ShapeDtypeStruct((2, 1568, 2), _F32),
            jax.ShapeDtypeStruct((2, 1568, 4), _F32),
            jax.ShapeDtypeStruct((2, 1568, 4), _F32),
        ],
        scratch_shapes=[pltpu.VMEM((1600, 256), _BF)],
        compiler_params=pltpu.CompilerParams(
            dimension_semantics=("parallel",),
        ),
        name="rpn_decode",
    )(f_wh, anc_big, rpn_wt, rpn_b.reshape(1, 32),
      cls_wt, cls_b.reshape(1, 2), reg_wt, reg_b.reshape(1, 4))

    scores = sc.reshape(B, 196, 2)
    deltas = dl.reshape(B, 196, 4)
    boxes32 = bxo.reshape(2 * B, 98, 4)

    w1t = mconv1_w.transpose(2, 3, 1, 0).reshape(9, 256, 256).astype(_BF)
    wdt = mdeconv_w.transpose(2, 3, 0, 1).reshape(4, 256, 256).astype(_BF)
    wft = mfinal_w[:, :, 0, 0].T.astype(_BF)

    masks8 = pl.pallas_call(
        _mask_body,
        grid=(2 * B,),
        in_specs=[
            pl.BlockSpec((1, 196, 256), lambda t: (t // 2, 0, 0)),
            pl.BlockSpec((1, 98, 4), lambda t: (t, 0, 0)),
            pl.BlockSpec((9, 256, 256), lambda t: (0, 0, 0)),
            pl.BlockSpec((4, 256, 256), lambda t: (0, 0, 0)),
            pl.BlockSpec((256, 2), lambda t: (0, 0)),
        ],
        out_specs=pl.BlockSpec((1, 6272, 8), lambda t: (t, 0, 0)),
        out_shape=jax.ShapeDtypeStruct((2 * B, 6272, 8), _F32),
        scratch_shapes=[pltpu.VMEM((6304, 256), _BF)],
        compiler_params=pltpu.CompilerParams(
            dimension_semantics=("parallel",),
            vmem_limit_bytes=56 * 1024 * 1024,
        ),
        name="roialign_mask_head",
    )(f_hw, boxes32, w1t, wdt, wft)

    m = masks8.reshape(B, 2, 98, 8, 8, 2, 2, 2)[:, :, :, :7, :7]
    masks = m.transpose(0, 1, 2, 7, 3, 5, 4, 6).reshape(B, 196, 2, 14, 14)
    return (scores, deltas, anchors, masks)


# speedup vs baseline: 3.0896x; 1.0005x over previous
import numpy as np
import jax
import jax.numpy as jnp
from jax import lax
from jax.experimental import pallas as pl
from jax.experimental.pallas import tpu as pltpu

_BF = jnp.bfloat16
_F32 = jnp.float32


def _conv(x, w, stride, pad):
    return lax.conv_general_dilated(x.astype(_BF), w.transpose(2, 3, 1, 0).astype(_BF),
                                    (stride, stride), [(pad, pad), (pad, pad)],
                                    dimension_numbers=('NHWC', 'HWIO', 'NHWC'),
                                    preferred_element_type=_F32)


def _bn(x, p):
    inv = p['g'] / jnp.sqrt(p['v'] + 1e-5)
    return x * inv + (p['b'] - p['m'] * inv)


def _resblock(x, p, stride):
    out = jax.nn.relu(_bn(_conv(x, p['c1'], stride, 1), p['b1']))
    out = _bn(_conv(out, p['c2'], 1, 1), p['b2'])
    idn = _bn(_conv(x, p['dc'], stride, 0), p['db']) if 'dc' in p else x
    return jax.nn.relu(out + idn)


def _backbone(x, P):
    x = x.transpose(0, 2, 3, 1)
    x = jax.nn.relu(_bn(_conv(x, P['conv1'], 2, 3), P['bn1']))
    x = lax.reduce_window(x, -jnp.inf, lax.max, (1, 3, 3, 1), (1, 2, 2, 1),
                          [(0, 0), (1, 1), (1, 1), (0, 0)])
    for name, s in (('l1b1', 1), ('l1b2', 1), ('l2b1', 2), ('l2b2', 1), ('l3b1', 2), ('l3b2', 1)):
        x = _resblock(x, P[name], s)
    return x


def _anchor_grid_np():
    base, scale = 16, 2
    g = 224 // base
    ys, xs = np.meshgrid(np.arange(g), np.arange(g), indexing='ij')
    xc = base // 2 + xs * base
    yc = base // 2 + ys * base
    w = h = int(base * scale)
    a = np.stack([xc - w / 2, yc - h / 2, xc + w / 2, yc + h / 2], -1).astype(np.float32)
    return np.clip(a - 1, 0, None).reshape(-1, 4)


def _rpn_body(fm_ref, anc_ref, w1_ref, b1_ref, cw_ref, cb_ref, rw_ref, rb_ref,
              sc_ref, dl_ref, bx_ref, pad_ref):
    R = 1568
    pad_ref[0:16, :] = jnp.zeros((16, 256), _BF)
    pad_ref[R + 16:R + 32, :] = jnp.zeros((16, 256), _BF)
    pad_ref[16:R + 16, :] = fm_ref[...]
    r = lax.broadcasted_iota(jnp.int32, (R, 1), 0)
    loc = r % 196
    w_i = loc // 14
    h_i = loc % 14
    acc = jnp.zeros((R, 32), _F32)
    for k in range(9):
        kh, kw = k // 3, k % 3
        dh, dw = kh - 1, kw - 1
        s = dw * 14 + dh
        ok = (h_i + dh >= 0) & (h_i + dh <= 13) & (w_i + dw >= 0) & (w_i + dw <= 13)
        x = jnp.where(ok, pad_ref[16 + s:16 + s + R, :], jnp.zeros((), _BF))
        acc = acc + jnp.dot(x, w1_ref[k], preferred_element_type=_F32)
    rpn = jax.nn.relu(acc + b1_ref[...]).astype(_BF)
    sc = jnp.dot(rpn, cw_ref[...], preferred_element_type=_F32) + cb_ref[...]
    dl = jnp.dot(rpn, rw_ref[...], preferred_element_type=_F32) + rb_ref[...]
    sc_ref[0] = sc
    dl_ref[0] = dl
    anc = anc_ref[...]
    x_a = anc[:, 0:1]
    y_a = anc[:, 1:2]
    w_a = anc[:, 2:3] - x_a
    h_a = anc[:, 3:4] - y_a
    bx1 = dl[:, 0:1] * w_a + x_a
    by1 = dl[:, 1:2] * h_a + y_a
    bx2 = bx1 + jnp.exp(dl[:, 2:3]) * w_a
    by2 = by1 + jnp.exp(dl[:, 3:4]) * h_a
    b = jnp.concatenate([by1, bx1, by2, bx2], axis=1)
    bx_ref[0] = jnp.floor(jnp.clip(b, 0.0, 223.0)) / 223.0


def _mask_body(fm_ref, box_ref, w1_ref, wd_ref, wf_ref, out_ref, pad_ref):
    N = 98
    M = N * 64
    bx = box_ref[0].reshape(N, 1, 4)
    q = lax.broadcasted_iota(jnp.int32, (N, 8, 196), 2)
    hq = (q // 14).astype(_F32)
    wq = (q % 14).astype(_F32)
    ii = lax.broadcasted_iota(jnp.int32, (N, 8, 196), 1).astype(_F32)

    def taps(c1, c2, grid_f):
        t = c1 * 13.0 + ii * (c2 - c1) * 13.0 / 6.0
        t0 = jnp.floor(t)
        lt = t - t0
        wt = (jnp.where(grid_f == t0, 1.0 - lt, 0.0)
              + jnp.where(grid_f == jnp.ceil(t), lt, 0.0))
        return jnp.where(ii < 7.0, wt, 0.0)

    wy = taps(bx[:, :, 0:1], bx[:, :, 2:3], hq)
    wx = taps(bx[:, :, 1:2], bx[:, :, 3:4], wq)
    bigk = (wy[:, :, None, :] * wx[:, None, :, :]).astype(_BF).reshape(M, 196)
    crop = jnp.dot(bigk, fm_ref[0], preferred_element_type=_F32)
    pad_ref[0:16, :] = jnp.zeros((16, 256), _BF)
    pad_ref[M + 16:M + 32, :] = jnp.zeros((16, 256), _BF)
    pad_ref[16:M + 16, :] = crop.astype(_BF)
    acc = jnp.zeros((M, 256), _F32)
    for k in range(9):
        s = (k // 3 - 1) * 8 + (k % 3 - 1)
        acc = acc + jnp.dot(pad_ref[16 + s:16 + s + M, :], w1_ref[k],
                            preferred_element_type=_F32)
    xb = jax.nn.relu(acc).astype(_BF)
    zs = []
    for dij in range(4):
        h = jnp.dot(xb, wd_ref[dij], preferred_element_type=_F32)
        hb = jax.nn.relu(h).astype(_BF)
        z = jnp.dot(hb, wf_ref[...], preferred_element_type=_F32)
        zs.append(1.0 / (1.0 + jnp.exp(-z)))
    out_ref[0] = jnp.concatenate(zs, axis=1)


def kernel(x, backbone_params, rpn_w, rpn_b, cls_w, cls_b, reg_w, reg_b,
           mconv1_w, mdeconv_w, mfinal_w):
    B = x.shape[0]
    fmap = _backbone(x, backbone_params)

    f_wh = fmap.transpose(0, 2, 1, 3).reshape(B * 196, 256).astype(_BF)
    f_hw = fmap.reshape(B, 196, 256).astype(_BF)

    anchors = jnp.asarray(_anchor_grid_np())
    anc_big = jnp.tile(anchors, (8, 1))

    rpn_wt = rpn_w.transpose(2, 3, 1, 0).reshape(9, 256, 32).astype(_BF)
    cls_wt = cls_w[:, :, 0, 0].T.astype(_BF)
    reg_wt = reg_w[:, :, 0, 0].T.astype(_BF)

    sc, dl, bxo = pl.pallas_call(
        _rpn_body,
        grid=(2,),
        in_specs=[
            pl.BlockSpec((1568, 256), lambda g: (g, 0)),
            pl.BlockSpec((1568, 4), lambda g: (0, 0)),
            pl.BlockSpec((9, 256, 32), lambda g: (0, 0, 0)),
            pl.BlockSpec((1, 32), lambda g: (0, 0)),
            pl.BlockSpec((32, 2), lambda g: (0, 0)),
            pl.BlockSpec((1, 2), lambda g: (0, 0)),
            pl.BlockSpec((32, 4), lambda g: (0, 0)),
            pl.BlockSpec((1, 4), lambda g: (0, 0)),
        ],
        out_specs=[
            pl.BlockSpec((1, 1568, 2), lambda g: (g, 0, 0)),
            pl.BlockSpec((1, 1568, 4), lambda g: (g, 0, 0)),
            pl.BlockSpec((1, 1568, 4), lambda g: (g, 0, 0)),
        ],
        out_shape=[
            jax.ShapeDtypeStruct((2, 1568, 2), _F32),
            jax.ShapeDtypeStruct((2, 1568, 4), _F32),
            jax.ShapeDtypeStruct((2, 1568, 4), _F32),
        ],
        scratch_shapes=[pltpu.VMEM((1600, 256), _BF)],
        compiler_params=pltpu.CompilerParams(
            dimension_semantics=("arbitrary",),
        ),
        name="rpn_decode",
    )(f_wh, anc_big, rpn_wt, rpn_b.reshape(1, 32),
      cls_wt, cls_b.reshape(1, 2), reg_wt, reg_b.reshape(1, 4))

    scores = sc.reshape(B, 196, 2)
    deltas = dl.reshape(B, 196, 4)
    boxes32 = bxo.reshape(2 * B, 98, 4)

    w1t = mconv1_w.transpose(2, 3, 1, 0).reshape(9, 256, 256).astype(_BF)
    wdt = mdeconv_w.transpose(2, 3, 0, 1).reshape(4, 256, 256).astype(_BF)
    wft = mfinal_w[:, :, 0, 0].T.astype(_BF)

    masks8 = pl.pallas_call(
        _mask_body,
        grid=(2 * B,),
        in_specs=[
            pl.BlockSpec((1, 196, 256), lambda t: (t // 2, 0, 0)),
            pl.BlockSpec((1, 98, 4), lambda t: (t, 0, 0)),
            pl.BlockSpec((9, 256, 256), lambda t: (0, 0, 0)),
            pl.BlockSpec((4, 256, 256), lambda t: (0, 0, 0)),
            pl.BlockSpec((256, 2), lambda t: (0, 0)),
        ],
        out_specs=pl.BlockSpec((1, 6272, 8), lambda t: (t, 0, 0)),
        out_shape=jax.ShapeDtypeStruct((2 * B, 6272, 8), _F32),
        scratch_shapes=[pltpu.VMEM((6304, 256), _BF)],
        compiler_params=pltpu.CompilerParams(
            dimension_semantics=("arbitrary",),
            vmem_limit_bytes=56 * 1024 * 1024,
        ),
        name="roialign_mask_head",
    )(f_hw, boxes32, w1t, wdt, wft)

    m = masks8.reshape(B, 2, 98, 8, 8, 2, 2, 2)[:, :, :, :7, :7]
    masks = m.transpose(0, 1, 2, 7, 3, 5, 4, 6).reshape(B, 196, 2, 14, 14)
    return (scores, deltas, anchors, masks)
